# baseline (speedup 1.0000x reference)
import jax
import jax.numpy as jnp
from jax import lax
from jax.experimental import pallas as pl
from jax.experimental.pallas import tpu as pltpu
from jax.experimental.pallas import tpu_sc as plsc

NC = 2
NS = 16
NW = NC * NS

HIDDEN = 64
CHUNK = 128


def _gather_kernel(n_total):
    n_chunks = n_total // CHUNK
    per_w = n_chunks // NW
    assert per_w * NW == n_chunks and per_w % 2 == 0

    mesh = plsc.VectorSubcoreMesh(core_axis_name="c", subcore_axis_name="s")

    @pl.kernel(
        mesh=mesh,
        compiler_params=pltpu.CompilerParams(use_tc_tiling_on_sc=False),
        out_type=jax.ShapeDtypeStruct((n_total, HIDDEN), jnp.float32),
        scratch_types=[
            pltpu.VMEM((per_w, CHUNK), jnp.int32),
            pltpu.VMEM((CHUNK, HIDDEN), jnp.float32),
            pltpu.VMEM((CHUNK, HIDDEN), jnp.float32),
            pltpu.SemaphoreType.DMA,
            pltpu.SemaphoreType.DMA,
            pltpu.SemaphoreType.DMA,
            pltpu.SemaphoreType.DMA,
        ],
    )
    def k(idx_hbm, table_hbm, out_hbm, idx_v, buf0, buf1, g0, g1, w0, w1):
        wid = lax.axis_index("s") * NC + lax.axis_index("c")
        base = wid * per_w
        pltpu.sync_copy(idx_hbm.at[wid], idx_v)

        def step(i, _):
            j0 = 2 * i
            j1 = j0 + 1
            c0 = pltpu.async_copy(table_hbm.at[idx_v.at[j0]], buf0, g0)
            c1 = pltpu.async_copy(table_hbm.at[idx_v.at[j1]], buf1, g1)
            c0.wait()
            o0 = pltpu.async_copy(
                buf0, out_hbm.at[pl.ds((base + j0) * CHUNK, CHUNK)], w0)
            c1.wait()
            o1 = pltpu.async_copy(
                buf1, out_hbm.at[pl.ds((base + j1) * CHUNK, CHUNK)], w1)
            o0.wait()
            o1.wait()
            return 0

        lax.fori_loop(0, per_w // 2, step, 0)

    return k


def kernel(batch, positions, mask, table):
    del positions, mask
    B, L = batch.shape
    n_total = B * L
    idx = batch.reshape(NW, n_total // (NW * CHUNK), CHUNK).astype(jnp.int32)
    out = _gather_kernel(n_total)(idx, table)
    return out.reshape(B, L, HIDDEN)


# speedup vs baseline: 1.0186x; 1.0186x over previous
import jax
import jax.numpy as jnp
from jax import lax
from jax.experimental import pallas as pl
from jax.experimental.pallas import tpu as pltpu
from jax.experimental.pallas import tpu_sc as plsc

NC = 2
NS = 16
NW = NC * NS

HIDDEN = 64
CHUNK = 128
K = 10


def _gather_kernel(n_total):
    n_chunks = n_total // CHUNK
    per_w = n_chunks // NW
    supers = per_w // K
    assert per_w * NW == n_chunks and supers * K == per_w

    mesh = plsc.VectorSubcoreMesh(core_axis_name="c", subcore_axis_name="s")

    @pl.kernel(
        mesh=mesh,
        compiler_params=pltpu.CompilerParams(use_tc_tiling_on_sc=False),
        out_type=jax.ShapeDtypeStruct((n_total, HIDDEN), jnp.float32),
        scratch_types=(
            [pltpu.VMEM((per_w, CHUNK), jnp.int32)]
            + [pltpu.VMEM((CHUNK, HIDDEN), jnp.float32)] * K
            + [pltpu.SemaphoreType.DMA] * (2 * K)
        ),
    )
    def k(idx_hbm, table_hbm, out_hbm, idx_v, *rest):
        bufs = rest[:K]
        gsems = rest[K:2 * K]
        wsems = rest[2 * K:3 * K]
        wid = lax.axis_index("s") * NC + lax.axis_index("c")
        base = wid * per_w
        pltpu.sync_copy(idx_hbm.at[wid], idx_v)

        for b in range(K):
            pltpu.async_copy(table_hbm.at[idx_v.at[b]], bufs[b], gsems[b])

        def super_step(s, _):
            for b in range(K):
                j = s * K + b
                dst = out_hbm.at[pl.ds((base + j) * CHUNK, CHUNK)]
                pltpu.make_async_copy(
                    table_hbm.at[idx_v.at[j]], bufs[b], gsems[b]).wait()
                pltpu.async_copy(bufs[b], dst, wsems[b])

                @pl.when(s < supers - 1)
                def _prefetch(b=b, j=j, dst=dst):
                    pltpu.make_async_copy(bufs[b], dst, wsems[b]).wait()
                    pltpu.async_copy(
                        table_hbm.at[idx_v.at[j + K]], bufs[b], gsems[b])
            return 0

        lax.fori_loop(0, supers, super_step, 0)

        for b in range(K):
            drain_dst = out_hbm.at[pl.ds(base * CHUNK, CHUNK)]
            pltpu.make_async_copy(bufs[b], drain_dst, wsems[b]).wait()

    return k


def kernel(batch, positions, mask, table):
    del positions, mask
    B, L = batch.shape
    n_total = B * L
    idx = batch.reshape(NW, n_total // (NW * CHUNK), CHUNK).astype(jnp.int32)
    out = _gather_kernel(n_total)(idx, table)
    return out.reshape(B, L, HIDDEN)
